# baseline (speedup 1.0000x reference)
import functools

import numpy as np
import jax
import jax.numpy as jnp
from jax import lax
from jax.experimental import pallas as pl
from jax.experimental.pallas import tpu as pltpu, tpu_sc as plsc

_S = 512
_C = 6
_P = 8
_NW = 32


def _proj_body(x_ref, W_ref, ones_ref, y_ref):
    x = x_ref[...]
    y = lax.dot_general(x.astype(jnp.bfloat16), W_ref[...],
                        (((1,), (1,)), ((), ())),
                        preferred_element_type=jnp.float32)
    y_ref[...] = y + ones_ref[...]


def _make_sc_segment_sum(n):
    rows_w = n // _NW
    vals_w = rows_w * _P
    acc_n = _S * _P
    mesh = plsc.VectorSubcoreMesh(core_axis_name="c", subcore_axis_name="s",
                                  num_cores=2, num_subcores=16)

    @functools.partial(
        pl.kernel,
        out_type=jax.ShapeDtypeStruct((_NW, acc_n), jnp.float32),
        mesh=mesh,
        scratch_types=[
            pltpu.VMEM((vals_w,), jnp.float32),
            pltpu.VMEM((rows_w,), jnp.int32),
            pltpu.VMEM((2 * acc_n,), jnp.float32),
        ],
        compiler_params=pltpu.CompilerParams(needs_layout_passes=False),
    )
    def sc(y_hbm, ids_hbm, out_hbm, y_v, ids_v, acc_v):
        wid = lax.axis_index("c") * 16 + lax.axis_index("s")
        base = wid * rows_w
        pltpu.sync_copy(y_hbm.at[pl.ds(base * _P, vals_w)], y_v)
        pltpu.sync_copy(ids_hbm.at[pl.ds(base, rows_w)], ids_v)

        zeros16 = jnp.zeros((16,), jnp.float32)

        def _zero(k, c):
            acc_v[pl.ds(k * 16, 16)] = zeros16
            return c

        lax.fori_loop(0, (2 * acc_n) // 16, _zero, 0)

        lanes = lax.iota(jnp.int32, 16)
        pat = lanes >> 3
        laneoff = (lanes & 7) + pat * acc_n

        def _step(j, c):
            rowsel = pat + 2 * j
            seg = plsc.load_gather(ids_v, [rowsel])
            vals = y_v[pl.ds(j * 16, 16)]
            tgt = (seg << 3) + laneoff
            plsc.addupdate_scatter(acc_v, [tgt], vals)
            return c

        lax.fori_loop(0, rows_w // 2, _step, 0)

        def _merge(k, c):
            v = acc_v[pl.ds(acc_n + k * 16, 16)]
            a = acc_v[pl.ds(k * 16, 16)]
            acc_v[pl.ds(k * 16, 16)] = a + v
            return c

        lax.fori_loop(0, acc_n // 16, _merge, 0)
        pltpu.sync_copy(acc_v.at[pl.ds(0, acc_n)], out_hbm.at[wid])

    return sc


def _fin_body(p_ref, bias_ref, out_ref):
    s = jnp.sum(p_ref[...], axis=0)
    cnt = jnp.clip(s[:, _C:_C + 1], 1.0, None)
    out_ref[...] = jax.nn.sigmoid(s / cnt + bias_ref[...])


def kernel(x, batch, W, b):
    n, d = x.shape
    r = 0
    for cand in range(128, 4097, 128):
        if n % cand == 0:
            r = cand
    nb = n // r

    Wp = jnp.zeros((_P, d), jnp.bfloat16).at[:_C].set(W.astype(jnp.bfloat16))
    ones8 = jnp.asarray((np.arange(_P) == _C).astype(np.float32).reshape(1, _P))

    y = pl.pallas_call(
        _proj_body,
        grid=(nb,),
        in_specs=[
            pl.BlockSpec((r, d), lambda i: (i, 0)),
            pl.BlockSpec((_P, d), lambda i: (0, 0)),
            pl.BlockSpec((1, _P), lambda i: (0, 0)),
        ],
        out_specs=pl.BlockSpec((r, _P), lambda i: (i, 0)),
        out_shape=jax.ShapeDtypeStruct((n, _P), jnp.float32),
    )(x, Wp, ones8)

    sc = _make_sc_segment_sum(n)
    part = sc(y.reshape(-1), batch.astype(jnp.int32))
    part3 = part.reshape(_NW, _S, _P)

    bias = jnp.concatenate([b, jnp.zeros((_P - _C,), b.dtype)]).reshape(1, _P)
    out8 = pl.pallas_call(
        _fin_body,
        in_specs=[
            pl.BlockSpec((_NW, _S, _P), lambda: (0, 0, 0)),
            pl.BlockSpec((1, _P), lambda: (0, 0)),
        ],
        out_specs=pl.BlockSpec((_S, _P), lambda: (0, 0)),
        out_shape=jax.ShapeDtypeStruct((_S, _P), jnp.float32),
    )(part3, bias)
    return out8[:, :_C]


# speedup vs baseline: 1.0048x; 1.0048x over previous
import functools

import numpy as np
import jax
import jax.numpy as jnp
from jax import lax
from jax.experimental import pallas as pl
from jax.experimental.pallas import tpu as pltpu, tpu_sc as plsc

_S = 512
_C = 6
_P = 8
_NW = 32


def _proj_body(x_ref, W_ref, ones_ref, y_ref):
    x = x_ref[...]
    y = lax.dot_general(x.astype(jnp.bfloat16), W_ref[...],
                        (((1,), (1,)), ((), ())),
                        preferred_element_type=jnp.float32)
    y_ref[...] = y + ones_ref[...]


def _make_sc_segment_sum(n):
    rows_w = n // _NW
    vals_w = rows_w * _P
    acc_n = _S * _P
    mesh = plsc.VectorSubcoreMesh(core_axis_name="c", subcore_axis_name="s",
                                  num_cores=2, num_subcores=16)

    @functools.partial(
        pl.kernel,
        out_type=jax.ShapeDtypeStruct((_NW, acc_n), jnp.float32),
        mesh=mesh,
        scratch_types=[
            pltpu.VMEM((vals_w,), jnp.float32),
            pltpu.VMEM((rows_w,), jnp.int32),
            pltpu.VMEM((2 * acc_n,), jnp.float32),
        ],
        compiler_params=pltpu.CompilerParams(needs_layout_passes=False),
    )
    def sc(y_hbm, ids_hbm, out_hbm, y_v, ids_v, acc_v):
        wid = lax.axis_index("c") * 16 + lax.axis_index("s")
        base = wid * rows_w
        pltpu.sync_copy(y_hbm.at[pl.ds(base * _P, vals_w)], y_v)
        pltpu.sync_copy(ids_hbm.at[pl.ds(base, rows_w)], ids_v)

        zeros16 = jnp.zeros((16,), jnp.float32)
        unroll = 8

        def _zero(k, c):
            for u in range(unroll):
                acc_v[pl.ds(k * (16 * unroll) + u * 16, 16)] = zeros16
            return c

        lax.fori_loop(0, (2 * acc_n) // (16 * unroll), _zero, 0)

        lanes = lax.iota(jnp.int32, 16)
        pat = lanes >> 3
        laneoff = (lanes & 7) + pat * acc_n

        def _step(jo, c):
            for u in range(unroll):
                rowsel = pat + jo * (2 * unroll) + 2 * u
                seg = plsc.load_gather(ids_v, [rowsel])
                vals = y_v[pl.ds(jo * (16 * unroll) + u * 16, 16)]
                tgt = (seg << 3) + laneoff
                plsc.addupdate_scatter(acc_v, [tgt], vals)
            return c

        lax.fori_loop(0, rows_w // (2 * unroll), _step, 0)

        def _merge(k, c):
            for u in range(unroll):
                o = k * (16 * unroll) + u * 16
                v = acc_v[pl.ds(acc_n + o, 16)]
                a = acc_v[pl.ds(o, 16)]
                acc_v[pl.ds(o, 16)] = a + v
            return c

        lax.fori_loop(0, acc_n // (16 * unroll), _merge, 0)
        pltpu.sync_copy(acc_v.at[pl.ds(0, acc_n)], out_hbm.at[wid])

    return sc


def _fin_body(p_ref, bias_ref, out_ref):
    s = jnp.sum(p_ref[...], axis=0)
    cnt = jnp.clip(s[:, _C:_C + 1], 1.0, None)
    out_ref[...] = jax.nn.sigmoid(s / cnt + bias_ref[...])


def kernel(x, batch, W, b):
    n, d = x.shape
    r = 0
    for cand in range(128, 4097, 128):
        if n % cand == 0:
            r = cand
    nb = n // r

    Wp = jnp.zeros((_P, d), jnp.bfloat16).at[:_C].set(W.astype(jnp.bfloat16))
    ones8 = jnp.asarray((np.arange(_P) == _C).astype(np.float32).reshape(1, _P))

    y = pl.pallas_call(
        _proj_body,
        grid=(nb,),
        in_specs=[
            pl.BlockSpec((r, d), lambda i: (i, 0)),
            pl.BlockSpec((_P, d), lambda i: (0, 0)),
            pl.BlockSpec((1, _P), lambda i: (0, 0)),
        ],
        out_specs=pl.BlockSpec((r, _P), lambda i: (i, 0)),
        out_shape=jax.ShapeDtypeStruct((n, _P), jnp.float32),
    )(x, Wp, ones8)

    sc = _make_sc_segment_sum(n)
    part = sc(y.reshape(-1), batch.astype(jnp.int32))
    part3 = part.reshape(_NW, _S, _P)

    bias = jnp.concatenate([b, jnp.zeros((_P - _C,), b.dtype)]).reshape(1, _P)
    out8 = pl.pallas_call(
        _fin_body,
        in_specs=[
            pl.BlockSpec((_NW, _S, _P), lambda: (0, 0, 0)),
            pl.BlockSpec((1, _P), lambda: (0, 0)),
        ],
        out_specs=pl.BlockSpec((_S, _P), lambda: (0, 0)),
        out_shape=jax.ShapeDtypeStruct((_S, _P), jnp.float32),
    )(part3, bias)
    return out8[:, :_C]


# speedup vs baseline: 1.7167x; 1.7085x over previous
import functools

import numpy as np
import jax
import jax.numpy as jnp
from jax import lax
from jax.experimental import pallas as pl
from jax.experimental.pallas import tpu as pltpu, tpu_sc as plsc

_S = 512
_C = 6
_P = 8
_NW = 32
_NL = 16
_CH = 3328


def _proj_body(x_ref, W_ref, y_ref):
    x = x_ref[...]
    y_ref[...] = lax.dot_general(W_ref[...], x.astype(jnp.bfloat16),
                                 (((1,), (1,)), ((), ())),
                                 preferred_element_type=jnp.float32)


def _make_sc_segment_sum(n):
    units = n // 128
    per_w = units // _NW
    rem = units % _NW
    rows_w = per_w * 128
    nchunks = (rows_w + _CH - 1) // _CH
    assert rows_w % _CH == 0 and _CH % 128 == 0
    bank = _P * _S
    mesh = plsc.VectorSubcoreMesh(core_axis_name="c", subcore_axis_name="s",
                                  num_cores=2, num_subcores=16)

    @functools.partial(
        pl.kernel,
        out_type=jax.ShapeDtypeStruct((_NW, _C + 1, _S), jnp.float32),
        mesh=mesh,
        scratch_types=[
            pltpu.VMEM((_P, _CH), jnp.float32),
            pltpu.VMEM((rows_w,), jnp.int32),
            pltpu.VMEM((_P, 128), jnp.float32),
            pltpu.VMEM((128,), jnp.int32),
            pltpu.VMEM((_NL * bank,), jnp.float32),
            pltpu.VMEM((_C + 1, _S), jnp.float32),
        ],
        compiler_params=pltpu.CompilerParams(needs_layout_passes=False),
    )
    def sc(y_hbm, ids_hbm, out_hbm, y_v, ids_v, y2_v, ids2_v, acc_v, m_v):
        wid = lax.axis_index("c") * 16 + lax.axis_index("s")
        base = wid * rows_w
        pltpu.sync_copy(ids_hbm.at[pl.ds(base, rows_w)], ids_v)

        zeros16 = jnp.zeros((16,), jnp.float32)
        ones16 = jnp.ones((16,), jnp.float32)

        def _zero(k, c):
            for u in range(8):
                acc_v[pl.ds(k * 128 + u * 16, 16)] = zeros16
            return c

        lax.fori_loop(0, (_NL * bank) // 128, _zero, 0)

        slotbase = lax.iota(jnp.int32, 16) * bank

        for chunk in range(nchunks):
            cb = chunk * _CH
            pltpu.sync_copy(y_hbm.at[:, pl.ds(base + cb, _CH)], y_v)

            def _step(g, c):
                ids16 = ids_v[pl.ds(cb + g * 16, 16)]
                t0 = ids16 + slotbase
                for ch in range(_C):
                    vals = y_v[ch, pl.ds(g * 16, 16)]
                    plsc.addupdate_scatter(acc_v, [t0 + ch * _S], vals)
                plsc.addupdate_scatter(acc_v, [t0 + _C * _S], ones16)
                return c

            lax.fori_loop(0, _CH // 16, _step, 0)

        if rem:
            @pl.when(wid < rem)
            def _tail():
                tbase = _NW * rows_w + wid * 128
                pltpu.sync_copy(y_hbm.at[:, pl.ds(tbase, 128)], y2_v)
                pltpu.sync_copy(ids_hbm.at[pl.ds(tbase, 128)], ids2_v)

                def _step2(g, c):
                    ids16 = ids2_v[pl.ds(g * 16, 16)]
                    t0 = ids16 + slotbase
                    for ch in range(_C):
                        vals = y2_v[ch, pl.ds(g * 16, 16)]
                        plsc.addupdate_scatter(acc_v, [t0 + ch * _S], vals)
                    plsc.addupdate_scatter(acc_v, [t0 + _C * _S], ones16)
                    return c

                lax.fori_loop(0, 8, _step2, 0)

        def _merge(s, c):
            for ch in range(_C + 1):
                o = ch * _S + s * 16
                tot = acc_v[pl.ds(o, 16)]
                for l in range(1, _NL):
                    tot = tot + acc_v[pl.ds(l * bank + o, 16)]
                m_v[ch, pl.ds(s * 16, 16)] = tot
            return c

        lax.fori_loop(0, _S // 16, _merge, 0)
        pltpu.sync_copy(m_v, out_hbm.at[wid])

    return sc


def _fin_body(p_ref, bias_ref, out_ref):
    s = jnp.sum(p_ref[...], axis=0)
    cnt = jnp.clip(s[_C:_C + 1, :], 1.0, None)
    out_ref[...] = jax.nn.sigmoid(s[:_C, :] / cnt + bias_ref[...])


def kernel(x, batch, W, b):
    n, d = x.shape
    r = 0
    for cand in range(128, 4097, 128):
        if n % cand == 0:
            r = cand
    nb = n // r

    Wp = jnp.zeros((_P, d), jnp.bfloat16).at[:_C].set(W.astype(jnp.bfloat16))

    y_t = pl.pallas_call(
        _proj_body,
        grid=(nb,),
        in_specs=[
            pl.BlockSpec((r, d), lambda i: (i, 0)),
            pl.BlockSpec((_P, d), lambda i: (0, 0)),
        ],
        out_specs=pl.BlockSpec((_P, r), lambda i: (0, i)),
        out_shape=jax.ShapeDtypeStruct((_P, n), jnp.float32),
    )(x, Wp)

    sc = _make_sc_segment_sum(n)
    part = sc(y_t, batch.astype(jnp.int32))

    bias = b.reshape(_C, 1)
    out = pl.pallas_call(
        _fin_body,
        in_specs=[
            pl.BlockSpec((_NW, _C + 1, _S), lambda: (0, 0, 0)),
            pl.BlockSpec((_C, 1), lambda: (0, 0)),
        ],
        out_specs=pl.BlockSpec((_C, _S), lambda: (0, 0)),
        out_shape=jax.ShapeDtypeStruct((_C, _S), jnp.float32),
    )(part, bias)
    return out.T


# speedup vs baseline: 2.2772x; 1.3265x over previous
import functools

import numpy as np
import jax
import jax.numpy as jnp
from jax import lax
from jax.experimental import pallas as pl
from jax.experimental.pallas import tpu as pltpu, tpu_sc as plsc

_S = 512
_C = 6
_P = 8
_NW = 32
_NL = 16
_CH = 3328


def _proj_body(x_ref, W_ref, y_ref):
    x = x_ref[...]
    y_ref[...] = lax.dot_general(W_ref[...], x.astype(jnp.bfloat16),
                                 (((1,), (1,)), ((), ())),
                                 preferred_element_type=jnp.float32)


def _make_sc_segment_sum(n):
    units = n // 128
    per_w = units // _NW
    rem = units % _NW
    rows_w = per_w * 128
    nchunks = (rows_w + _CH - 1) // _CH
    assert rows_w % _CH == 0 and _CH % 128 == 0
    bank = _P * _S + 1
    acc_sz = ((_NL * bank + 127) // 128) * 128
    mesh = plsc.VectorSubcoreMesh(core_axis_name="c", subcore_axis_name="s",
                                  num_cores=2, num_subcores=16)

    @functools.partial(
        pl.kernel,
        out_type=jax.ShapeDtypeStruct((_NW, _C + 1, _S), jnp.float32),
        mesh=mesh,
        scratch_types=[
            pltpu.VMEM((_P, _CH), jnp.float32),
            pltpu.VMEM((rows_w,), jnp.int32),
            pltpu.VMEM((_P, 128), jnp.float32),
            pltpu.VMEM((128,), jnp.int32),
            pltpu.VMEM((acc_sz,), jnp.float32),
            pltpu.VMEM((_C + 1, _S), jnp.float32),
        ],
        compiler_params=pltpu.CompilerParams(needs_layout_passes=False),
    )
    def sc(y_hbm, ids_hbm, out_hbm, y_v, ids_v, y2_v, ids2_v, acc_v, m_v):
        wid = lax.axis_index("c") * 16 + lax.axis_index("s")
        base = wid * rows_w
        pltpu.sync_copy(ids_hbm.at[pl.ds(base, rows_w)], ids_v)

        zeros16 = jnp.zeros((16,), jnp.float32)
        ones16 = jnp.ones((16,), jnp.float32)

        def _zero(k, c):
            for u in range(8):
                acc_v[pl.ds(k * 128 + u * 16, 16)] = zeros16
            return c

        lax.fori_loop(0, acc_sz // 128, _zero, 0)

        slotbase = lax.iota(jnp.int32, 16) * bank

        for chunk in range(nchunks):
            cb = chunk * _CH
            pltpu.sync_copy(y_hbm.at[:, pl.ds(base + cb, _CH)], y_v)

            def _step(g, c):
                ids16 = ids_v[pl.ds(cb + g * 16, 16)]
                t0 = ids16 + slotbase
                vals = [y_v[ch, pl.ds(g * 16, 16)] for ch in range(_C)]
                tgts = [t0 + ch * _S for ch in range(_C + 1)]
                for ch in range(_C):
                    plsc.addupdate_scatter(acc_v, [tgts[ch]], vals[ch])
                plsc.addupdate_scatter(acc_v, [tgts[_C]], ones16)
                return c

            lax.fori_loop(0, _CH // 16, _step, 0)

        if rem:
            @pl.when(wid < rem)
            def _tail():
                tbase = _NW * rows_w + wid * 128
                pltpu.sync_copy(y_hbm.at[:, pl.ds(tbase, 128)], y2_v)
                pltpu.sync_copy(ids_hbm.at[pl.ds(tbase, 128)], ids2_v)

                def _step2(g, c):
                    ids16 = ids2_v[pl.ds(g * 16, 16)]
                    t0 = ids16 + slotbase
                    vals = [y2_v[ch, pl.ds(g * 16, 16)] for ch in range(_C)]
                    tgts = [t0 + ch * _S for ch in range(_C + 1)]
                    for ch in range(_C):
                        plsc.addupdate_scatter(acc_v, [tgts[ch]], vals[ch])
                    plsc.addupdate_scatter(acc_v, [tgts[_C]], ones16)
                    return c

                lax.fori_loop(0, 8, _step2, 0)

        def _merge(s, c):
            for ch in range(_C + 1):
                o = ch * _S + s * 16
                tot = acc_v[pl.ds(o, 16)]
                for l in range(1, _NL):
                    tot = tot + acc_v[pl.ds(l * bank + o, 16)]
                m_v[ch, pl.ds(s * 16, 16)] = tot
            return c

        lax.fori_loop(0, _S // 16, _merge, 0)
        pltpu.sync_copy(m_v, out_hbm.at[wid])

    return sc


def _fin_body(p_ref, bias_ref, out_ref):
    s = jnp.sum(p_ref[...], axis=0)
    cnt = jnp.clip(s[_C:_C + 1, :], 1.0, None)
    out_ref[...] = jax.nn.sigmoid(s[:_C, :] / cnt + bias_ref[...])


def kernel(x, batch, W, b):
    n, d = x.shape
    r = 0
    for cand in range(128, 4097, 128):
        if n % cand == 0:
            r = cand
    nb = n // r

    Wp = jnp.zeros((_P, d), jnp.bfloat16).at[:_C].set(W.astype(jnp.bfloat16))

    y_t = pl.pallas_call(
        _proj_body,
        grid=(nb,),
        in_specs=[
            pl.BlockSpec((r, d), lambda i: (i, 0)),
            pl.BlockSpec((_P, d), lambda i: (0, 0)),
        ],
        out_specs=pl.BlockSpec((_P, r), lambda i: (0, i)),
        out_shape=jax.ShapeDtypeStruct((_P, n), jnp.float32),
    )(x, Wp)

    sc = _make_sc_segment_sum(n)
    part = sc(y_t, batch.astype(jnp.int32))

    bias = b.reshape(_C, 1)
    out = pl.pallas_call(
        _fin_body,
        in_specs=[
            pl.BlockSpec((_NW, _C + 1, _S), lambda: (0, 0, 0)),
            pl.BlockSpec((_C, 1), lambda: (0, 0)),
        ],
        out_specs=pl.BlockSpec((_C, _S), lambda: (0, 0)),
        out_shape=jax.ShapeDtypeStruct((_C, _S), jnp.float32),
    )(part, bias)
    return out.T


# speedup vs baseline: 2.2901x; 1.0056x over previous
import functools

import numpy as np
import jax
import jax.numpy as jnp
from jax import lax
from jax.experimental import pallas as pl
from jax.experimental.pallas import tpu as pltpu, tpu_sc as plsc

_S = 512
_C = 6
_P = 8
_NW = 32
_NL = 16


def _proj_body(x_ref, W_ref, y_ref):
    x = x_ref[...]
    y_ref[...] = lax.dot_general(W_ref[...], x.astype(jnp.bfloat16),
                                 (((1,), (1,)), ((), ())),
                                 preferred_element_type=jnp.float32)


def _make_sc_segment_sum(n):
    units = n // 128
    per_w = units // _NW
    rem = units % _NW
    rows_w = per_w * 128
    ch_rows = 128
    for cu in range(1, per_w + 1):
        if per_w % cu == 0 and cu * 128 <= 3328:
            ch_rows = cu * 128
    nchunks = rows_w // ch_rows
    bank = _P * _S + 1
    acc_sz = ((_NL * bank + 127) // 128) * 128
    mesh = plsc.VectorSubcoreMesh(core_axis_name="c", subcore_axis_name="s",
                                  num_cores=2, num_subcores=16)

    @functools.partial(
        pl.kernel,
        out_type=jax.ShapeDtypeStruct((_NW, _C + 1, _S), jnp.float32),
        mesh=mesh,
        scratch_types=[
            pltpu.VMEM((_P, ch_rows), jnp.float32),
            pltpu.VMEM((rows_w,), jnp.int32),
            pltpu.VMEM((_P, 128), jnp.float32),
            pltpu.VMEM((128,), jnp.int32),
            pltpu.VMEM((acc_sz,), jnp.float32),
            pltpu.VMEM((_C + 1, _S), jnp.float32),
        ],
        compiler_params=pltpu.CompilerParams(needs_layout_passes=False),
    )
    def sc(y_hbm, ids_hbm, out_hbm, y_v, ids_v, y2_v, ids2_v, acc_v, m_v):
        wid = lax.axis_index("c") * 16 + lax.axis_index("s")
        base = wid * rows_w
        pltpu.sync_copy(ids_hbm.at[pl.ds(base, rows_w)], ids_v)

        zeros16 = jnp.zeros((16,), jnp.float32)
        ones16 = jnp.ones((16,), jnp.float32)

        def _zero(k, c):
            for u in range(8):
                acc_v[pl.ds(k * 128 + u * 16, 16)] = zeros16
            return c

        lax.fori_loop(0, acc_sz // 128, _zero, 0)

        slotbase = lax.iota(jnp.int32, 16) * bank

        for chunk in range(nchunks):
            cb = chunk * ch_rows
            pltpu.sync_copy(y_hbm.at[:, pl.ds(base + cb, ch_rows)], y_v)

            def _step(g, c):
                ids16 = ids_v[pl.ds(cb + g * 16, 16)]
                t0 = ids16 + slotbase
                vals = [y_v[ch, pl.ds(g * 16, 16)] for ch in range(_C)]
                tgts = [t0 + ch * _S for ch in range(_C + 1)]
                for ch in range(_C):
                    plsc.addupdate_scatter(acc_v, [tgts[ch]], vals[ch])
                plsc.addupdate_scatter(acc_v, [tgts[_C]], ones16)
                return c

            lax.fori_loop(0, ch_rows // 16, _step, 0)

        if rem:
            @pl.when(wid < rem)
            def _tail():
                tbase = _NW * rows_w + wid * 128
                pltpu.sync_copy(y_hbm.at[:, pl.ds(tbase, 128)], y2_v)
                pltpu.sync_copy(ids_hbm.at[pl.ds(tbase, 128)], ids2_v)

                def _step2(g, c):
                    ids16 = ids2_v[pl.ds(g * 16, 16)]
                    t0 = ids16 + slotbase
                    vals = [y2_v[ch, pl.ds(g * 16, 16)] for ch in range(_C)]
                    tgts = [t0 + ch * _S for ch in range(_C + 1)]
                    for ch in range(_C):
                        plsc.addupdate_scatter(acc_v, [tgts[ch]], vals[ch])
                    plsc.addupdate_scatter(acc_v, [tgts[_C]], ones16)
                    return c

                lax.fori_loop(0, 8, _step2, 0)

        def _merge(s, c):
            for ch in range(_C + 1):
                o = ch * _S + s * 16
                tot = acc_v[pl.ds(o, 16)]
                for l in range(1, _NL):
                    tot = tot + acc_v[pl.ds(l * bank + o, 16)]
                m_v[ch, pl.ds(s * 16, 16)] = tot
            return c

        lax.fori_loop(0, _S // 16, _merge, 0)
        pltpu.sync_copy(m_v, out_hbm.at[wid])

    return sc


def _fin_body(p1_ref, p2_ref, bias_ref, out_ref):
    s = jnp.sum(p1_ref[...], axis=0) + jnp.sum(p2_ref[...], axis=0)
    cnt = jnp.clip(s[_C:_C + 1, :], 1.0, None)
    out_ref[...] = jax.nn.sigmoid(s[:_C, :] / cnt + bias_ref[...])


def kernel(x, batch, W, b):
    n, d = x.shape
    nh = n // 2
    r = 0
    for cand in range(128, 4097, 128):
        if nh % cand == 0:
            r = cand
    nbh = nh // r

    Wp = jnp.zeros((_P, d), jnp.bfloat16).at[:_C].set(W.astype(jnp.bfloat16))
    ids32 = batch.astype(jnp.int32)
    sc = _make_sc_segment_sum(nh)

    def proj_half(off):
        return pl.pallas_call(
            _proj_body,
            grid=(nbh,),
            in_specs=[
                pl.BlockSpec((r, d), lambda i: (i + off, 0)),
                pl.BlockSpec((_P, d), lambda i: (0, 0)),
            ],
            out_specs=pl.BlockSpec((_P, r), lambda i: (0, i)),
            out_shape=jax.ShapeDtypeStruct((_P, nh), jnp.float32),
        )(x, Wp)

    y1 = proj_half(0)
    part1 = sc(y1, ids32[:nh])
    y2 = proj_half(nbh)
    part2 = sc(y2, ids32[nh:])

    bias = b.reshape(_C, 1)
    out = pl.pallas_call(
        _fin_body,
        in_specs=[
            pl.BlockSpec((_NW, _C + 1, _S), lambda: (0, 0, 0)),
            pl.BlockSpec((_NW, _C + 1, _S), lambda: (0, 0, 0)),
            pl.BlockSpec((_C, 1), lambda: (0, 0)),
        ],
        out_specs=pl.BlockSpec((_C, _S), lambda: (0, 0)),
        out_shape=jax.ShapeDtypeStruct((_C, _S), jnp.float32),
    )(part1, part2, bias)
    return out.T


# speedup vs baseline: 2.3178x; 1.0121x over previous
import functools

import numpy as np
import jax
import jax.numpy as jnp
from jax import lax
from jax.experimental import pallas as pl
from jax.experimental.pallas import tpu as pltpu, tpu_sc as plsc

_S = 512
_C = 6
_P = 8
_NW = 32
_NL = 16


def _proj_body(x_ref, W_ref, y_ref):
    x = x_ref[...]
    y_ref[...] = lax.dot_general(W_ref[...], x.astype(jnp.bfloat16),
                                 (((1,), (1,)), ((), ())),
                                 preferred_element_type=jnp.float32)


def _make_sc_segment_sum(n, ids_off=0):
    units = n // 128
    per_w = units // _NW
    rem = units % _NW
    rows_w = per_w * 128
    ch_rows = 128
    for cu in range(1, per_w + 1):
        if per_w % cu == 0 and cu * 128 <= 3328:
            ch_rows = cu * 128
    nchunks = rows_w // ch_rows
    bank = _P * _S + 1
    acc_sz = ((_NL * bank + 127) // 128) * 128
    mesh = plsc.VectorSubcoreMesh(core_axis_name="c", subcore_axis_name="s",
                                  num_cores=2, num_subcores=16)

    @functools.partial(
        pl.kernel,
        out_type=jax.ShapeDtypeStruct((_NW, _C + 1, _S), jnp.float32),
        mesh=mesh,
        scratch_types=[
            pltpu.VMEM((_P, ch_rows), jnp.float32),
            pltpu.VMEM((rows_w,), jnp.int32),
            pltpu.VMEM((_P, 128), jnp.float32),
            pltpu.VMEM((128,), jnp.int32),
            pltpu.VMEM((acc_sz,), jnp.float32),
            pltpu.VMEM((_C + 1, _S), jnp.float32),
        ],
        compiler_params=pltpu.CompilerParams(needs_layout_passes=False),
    )
    def sc(y_hbm, ids_hbm, out_hbm, y_v, ids_v, y2_v, ids2_v, acc_v, m_v):
        wid = lax.axis_index("c") * 16 + lax.axis_index("s")
        base = wid * rows_w
        pltpu.sync_copy(ids_hbm.at[pl.ds(ids_off + base, rows_w)], ids_v)

        zeros16 = jnp.zeros((16,), jnp.float32)
        ones16 = jnp.ones((16,), jnp.float32)

        def _zero(k, c):
            for u in range(8):
                acc_v[pl.ds(k * 128 + u * 16, 16)] = zeros16
            return c

        lax.fori_loop(0, acc_sz // 128, _zero, 0)

        slotbase = lax.iota(jnp.int32, 16) * bank

        for chunk in range(nchunks):
            cb = chunk * ch_rows
            pltpu.sync_copy(y_hbm.at[:, pl.ds(base + cb, ch_rows)], y_v)

            def _step(g, c):
                ids16 = ids_v[pl.ds(cb + g * 16, 16)]
                t0 = ids16 + slotbase
                vals = [y_v[ch, pl.ds(g * 16, 16)] for ch in range(_C)]
                tgts = [t0 + ch * _S for ch in range(_C + 1)]
                for ch in range(_C):
                    plsc.addupdate_scatter(acc_v, [tgts[ch]], vals[ch])
                plsc.addupdate_scatter(acc_v, [tgts[_C]], ones16)
                return c

            lax.fori_loop(0, ch_rows // 16, _step, 0)

        if rem:
            @pl.when(wid < rem)
            def _tail():
                tbase = _NW * rows_w + wid * 128
                pltpu.sync_copy(y_hbm.at[:, pl.ds(tbase, 128)], y2_v)
                pltpu.sync_copy(ids_hbm.at[pl.ds(ids_off + tbase, 128)], ids2_v)

                def _step2(g, c):
                    ids16 = ids2_v[pl.ds(g * 16, 16)]
                    t0 = ids16 + slotbase
                    vals = [y2_v[ch, pl.ds(g * 16, 16)] for ch in range(_C)]
                    tgts = [t0 + ch * _S for ch in range(_C + 1)]
                    for ch in range(_C):
                        plsc.addupdate_scatter(acc_v, [tgts[ch]], vals[ch])
                    plsc.addupdate_scatter(acc_v, [tgts[_C]], ones16)
                    return c

                lax.fori_loop(0, 8, _step2, 0)

        def _merge(s, c):
            for ch in range(_C + 1):
                o = ch * _S + s * 16
                tot = acc_v[pl.ds(o, 16)]
                for l in range(1, _NL):
                    tot = tot + acc_v[pl.ds(l * bank + o, 16)]
                m_v[ch, pl.ds(s * 16, 16)] = tot
            return c

        lax.fori_loop(0, _S // 16, _merge, 0)
        pltpu.sync_copy(m_v, out_hbm.at[wid])

    return sc


_K = 4


def _fin_body(*refs):
    p_refs, bias_ref, out_ref = refs[:_K], refs[_K], refs[_K + 1]
    s = p_refs[0][...].sum(axis=0)
    for p in p_refs[1:]:
        s = s + p[...].sum(axis=0)
    cnt = jnp.clip(s[_C:_C + 1, :], 1.0, None)
    out_ref[...] = jax.nn.sigmoid(s[:_C, :] / cnt + bias_ref[...])


def kernel(x, batch, W, b):
    n, d = x.shape
    nq = n // _K
    r = 0
    for cand in range(128, 4097, 128):
        if nq % cand == 0:
            r = cand
    nbq = nq // r

    Wp = jnp.zeros((_P, d), jnp.bfloat16).at[:_C].set(W.astype(jnp.bfloat16))
    ids32 = batch.astype(jnp.int32)

    def proj_chunk(off):
        return pl.pallas_call(
            _proj_body,
            grid=(nbq,),
            in_specs=[
                pl.BlockSpec((r, d), lambda i: (i + off, 0)),
                pl.BlockSpec((_P, d), lambda i: (0, 0)),
            ],
            out_specs=pl.BlockSpec((_P, r), lambda i: (0, i)),
            out_shape=jax.ShapeDtypeStruct((_P, nq), jnp.float32),
        )(x, Wp)

    parts = []
    for q in range(_K):
        y_q = proj_chunk(q * nbq)
        sc_q = _make_sc_segment_sum(nq, ids_off=q * nq)
        parts.append(sc_q(y_q, ids32))

    bias = b.reshape(_C, 1)
    out = pl.pallas_call(
        _fin_body,
        in_specs=(
            [pl.BlockSpec((_NW, _C + 1, _S), lambda: (0, 0, 0))
             for _ in range(_K)]
            + [pl.BlockSpec((_C, 1), lambda: (0, 0))]
        ),
        out_specs=pl.BlockSpec((_C, _S), lambda: (0, 0)),
        out_shape=jax.ShapeDtypeStruct((_C, _S), jnp.float32),
    )(*parts, bias)
    return out.T
